# baseline (speedup 1.0000x reference)
import functools
import jax
import jax.numpy as jnp
from jax.experimental import pallas as pl
from jax.experimental.pallas import tpu as pltpu

NEG_INF = float("-inf")


def _router_body(x_ref, rw_ref, rb_ref, out_ref):
    lt = jax.lax.dot_general(
        rw_ref[...], x_ref[...], (((1,), (1,)), ((), ())),
        preferred_element_type=jnp.float32)
    out_ref[...] = lt + rb_ref[:, 0:1]


def _top2_row(v):
    n = v.shape[1]
    idx = jax.lax.broadcasted_iota(jnp.int32, v.shape, 1)
    big = jnp.int32(n)
    v1 = jnp.max(v)
    i1 = jnp.min(jnp.where(v == v1, idx, big))
    vm = jnp.where(idx == i1, NEG_INF, v)
    v2 = jnp.max(vm)
    i2 = jnp.min(jnp.where(vm == v2, idx, big))
    return v1, i1, v2, i2


def _routing_body(lt_ref, tok_ref, gate_ref):
    l0 = lt_ref[0:1, :]
    l1 = lt_ref[1:2, :]
    m = jnp.maximum(l0, l1)
    e0 = jnp.exp(l0 - m)
    e1 = jnp.exp(l1 - m)
    s = e0 + e1
    sm0 = e0 / s
    sm1 = e1 / s
    v00, t00, v01, t01 = _top2_row(sm0)
    v10, t10, v11, t11 = _top2_row(sm1)
    li = jax.lax.broadcasted_iota(jnp.int32, (1, 8), 1)
    tok = jnp.where(li == 0, t00,
          jnp.where(li == 1, t01,
          jnp.where(li == 2, t10,
          jnp.where(li == 3, t11, 0))))
    gate = jnp.where(li == 0, v00,
           jnp.where(li == 1, v10,
           jnp.where(li == 2, v01,
           jnp.where(li == 3, v11, 0.0))))
    tok_ref[...] = tok
    gate_ref[...] = gate


def _ffn_body(toks, x_ref, w_ref, b_ref, out_ref):
    j = pl.program_id(1)
    r = jax.lax.dot_general(
        x_ref[0], w_ref[...], (((1,), (1,)), ((), ())),
        preferred_element_type=jnp.float32)
    out_ref[pl.ds(j, 1), :] = r + b_ref[...]


def _combine_body(eout_ref, tok_ref, gate_ref, out_ref, *, tb):
    i = pl.program_id(0)
    rows = jax.lax.broadcasted_iota(jnp.int32, (tb, 8), 0) + i * tb
    a = jnp.where(rows == tok_ref[...], gate_ref[...], 0.0)
    out_ref[...] = jnp.dot(a, eout_ref[...], preferred_element_type=jnp.float32)


def kernel(x, router_w, router_b, expert_w, expert_b):
    b, s, h = x.shape
    e = router_w.shape[0]
    assert e == 2
    bs = b * s
    xf = x.reshape(bs, h)

    rw8 = jnp.zeros((8, h), jnp.float32).at[:e].set(router_w)
    rb8 = jnp.zeros((8, 128), jnp.float32).at[:e, 0].set(router_b)

    tb = 256
    n_t = bs // tb

    logits_t = pl.pallas_call(
        _router_body,
        grid=(n_t,),
        in_specs=[
            pl.BlockSpec((tb, h), lambda i: (i, 0)),
            pl.BlockSpec((8, h), lambda i: (0, 0)),
            pl.BlockSpec((8, 128), lambda i: (0, 0)),
        ],
        out_specs=pl.BlockSpec((8, tb), lambda i: (0, i)),
        out_shape=jax.ShapeDtypeStruct((8, bs), jnp.float32),
    )(xf, rw8, rb8)

    tokens, gates = pl.pallas_call(
        _routing_body,
        in_specs=[pl.BlockSpec((8, bs), lambda: (0, 0))],
        out_specs=[
            pl.BlockSpec((1, 8), lambda: (0, 0)),
            pl.BlockSpec((1, 8), lambda: (0, 0)),
        ],
        out_shape=[
            jax.ShapeDtypeStruct((1, 8), jnp.int32),
            jax.ShapeDtypeStruct((1, 8), jnp.float32),
        ],
    )(logits_t)

    wb = 256
    n_w = h // wb
    eb = expert_b.reshape(1, h)

    eout = pl.pallas_call(
        _ffn_body,
        grid_spec=pltpu.PrefetchScalarGridSpec(
            num_scalar_prefetch=1,
            grid=(n_w, 8),
            in_specs=[
                pl.BlockSpec((1, 1, h), lambda c, j, t: (t[j], 0, 0)),
                pl.BlockSpec((wb, h), lambda c, j, t: (c, 0)),
                pl.BlockSpec((1, wb), lambda c, j, t: (0, c)),
            ],
            out_specs=pl.BlockSpec((8, wb), lambda c, j, t: (0, c)),
        ),
        out_shape=jax.ShapeDtypeStruct((8, h), jnp.float32),
    )(tokens.reshape(8), xf.reshape(bs, 1, h), expert_w, eb)

    y = pl.pallas_call(
        functools.partial(_combine_body, tb=tb),
        grid=(n_t,),
        in_specs=[
            pl.BlockSpec((8, h), lambda i: (0, 0)),
            pl.BlockSpec((1, 8), lambda i: (0, 0)),
            pl.BlockSpec((1, 8), lambda i: (0, 0)),
        ],
        out_specs=pl.BlockSpec((tb, h), lambda i: (i, 0)),
        out_shape=jax.ShapeDtypeStruct((bs, h), jnp.float32),
    )(eout, tokens, gates)

    return y.reshape(b, s, h)
